# baseline (speedup 1.0000x reference)
import functools

import jax
import jax.numpy as jnp
from jax import lax
from jax.experimental import pallas as pl
from jax.experimental.pallas import tpu as pltpu
from jax.experimental.pallas import tpu_sc as plsc

B = 16384
IN = 64
H = 64
C = 64
M = 1000000
WD = 128
SB = 8192
HB = SB // 2

PG = -(-M // SB)
WR = PG * HB

NC = 2
NS = 16
NW = NC * NS
L = 16


@functools.cache
def _mesh():
    return plsc.VectorSubcoreMesh(
        core_axis_name="c", subcore_axis_name="s", num_cores=NC,
        num_subcores=NS)


def _wrow(ids):
    s = ids >> 13
    half = (ids >> 12) & 1
    p = (s << 12) | (ids & 4095)
    return p, half


def _pack_body(src_ref, dst_ref):
    t = jnp.transpose(src_ref[...], (1, 0))
    dst_ref[...] = jnp.concatenate([t[0:HB], t[HB:SB]], axis=1)


def _tc_pack(hidT):
    return pl.pallas_call(
        _pack_body,
        grid=(PG,),
        in_specs=[pl.BlockSpec((H, SB), lambda i: (0, i))],
        out_specs=pl.BlockSpec((HB, WD), lambda i: (i, 0)),
        out_shape=jax.ShapeDtypeStruct((WR, WD), jnp.float32),
    )(hidT)


def _unpack_body(src_ref, dst_ref):
    s = src_ref[...]
    t = jnp.concatenate([s[:, 0:H], s[:, H:WD]], axis=0)
    dst_ref[...] = jnp.transpose(t, (1, 0))


def _tc_unpack(w):
    return pl.pallas_call(
        _unpack_body,
        grid=(PG,),
        in_specs=[pl.BlockSpec((HB, WD), lambda i: (i, 0))],
        out_specs=pl.BlockSpec((H, SB), lambda i: (0, i)),
        out_shape=jax.ShapeDtypeStruct((H, M), jnp.float32),
    )(w)


GB = B // NW
GCH = 128
GNC = GB // GCH


def _sc_gather_body(w_hbm, idx_hbm, out_hbm, idx_v, pidx_v, rows_v, sem):
    wid = lax.axis_index("s") * NC + lax.axis_index("c")
    pltpu.sync_copy(idx_hbm.at[pl.ds(wid * GB, GB)], idx_v)
    for k in range(GB // L):
        v = idx_v[pl.ds(k * L, L)]
        p, _ = _wrow(v)
        pidx_v[pl.ds(k * L, L)] = p
    copies = [
        pltpu.async_copy(
            w_hbm.at[pidx_v.at[pl.ds(j * GCH, GCH)]],
            rows_v.at[pl.ds(j * GCH, GCH)], sem)
        for j in range(GNC)
    ]
    for cp in copies:
        cp.wait()
    pltpu.sync_copy(rows_v, out_hbm.at[pl.ds(wid * GB, GB)])


@functools.cache
def _sc_gather_kernel():
    return pl.kernel(
        _sc_gather_body,
        mesh=_mesh(),
        out_type=jax.ShapeDtypeStruct((B, WD), jnp.float32),
        scratch_types=[
            pltpu.VMEM((GB,), jnp.int32),
            pltpu.VMEM((GB,), jnp.int32),
            pltpu.VMEM((GB, WD), jnp.float32),
            pltpu.SemaphoreType.DMA,
        ],
        compiler_params=pltpu.CompilerParams(needs_layout_passes=False),
    )


RB = 2048


def _tc_gru_body(x_ref, hp_ref, ids_ref, wir, wiz, win, whr, whz, whn, br,
                 bz, bin_, bhn, wout, bout, newh_ref, logit_ref):
    x = x_ref[...]
    hp = hp_ref[...]
    upper = (ids_ref[...] & 4096) > 0
    h = jnp.where(upper, hp[:, H:WD], hp[:, 0:H])
    f32 = jnp.float32
    r = jax.nn.sigmoid(
        jnp.dot(x, wir[...], preferred_element_type=f32)
        + jnp.dot(h, whr[...], preferred_element_type=f32) + br[...])
    z = jax.nn.sigmoid(
        jnp.dot(x, wiz[...], preferred_element_type=f32)
        + jnp.dot(h, whz[...], preferred_element_type=f32) + bz[...])
    n = jnp.tanh(
        jnp.dot(x, win[...], preferred_element_type=f32) + bin_[...]
        + r * (jnp.dot(h, whn[...], preferred_element_type=f32) + bhn[...]))
    nh = (1.0 - z) * n + z * h
    newh_ref[...] = jnp.concatenate(
        [nh, jnp.zeros((RB, WD - H), f32)], axis=1)
    logit_ref[...] = (
        jnp.dot(nh, wout[...], preferred_element_type=f32) + bout[...])


def _tc_gru(x, hp, ids2, wir, wiz, win, whr, whz, whn, br, bz, bin_, bhn,
            wout, bout):
    xsp = pl.BlockSpec((RB, H), lambda i: (i, 0))
    hsp = pl.BlockSpec((RB, WD), lambda i: (i, 0))
    isp = pl.BlockSpec((RB, 1), lambda i: (i, 0))
    wsp = pl.BlockSpec((H, H), lambda i: (0, 0))
    bsp = pl.BlockSpec((1, H), lambda i: (0, 0))
    return pl.pallas_call(
        _tc_gru_body,
        grid=(B // RB,),
        in_specs=[xsp, hsp, isp, wsp, wsp, wsp, wsp, wsp, wsp, bsp, bsp,
                  bsp, bsp, wsp, bsp],
        out_specs=[hsp, xsp],
        out_shape=[
            jax.ShapeDtypeStruct((B, WD), jnp.float32),
            jax.ShapeDtypeStruct((B, C), jnp.float32),
        ],
    )(x, hp, ids2, wir, wiz, win, whr, whz, whn, br, bz, bin_, bhn, wout,
      bout)


WRNG = WR // NW
NV = WRNG // L
SEL = 2064
SCH = 16


def _sc_scatter_body(idx_hbm, newh_hbm, w_ref, idx_all, winner, posA, rowA,
                     posB, rowB, nbuf, wbuf, gsem, ssem):
    wid = lax.axis_index("s") * NC + lax.axis_index("c")
    lo = wid * WRNG
    iota = lax.iota(jnp.int32, L)

    pltpu.sync_copy(idx_hbm, idx_all)

    minus1 = jnp.full((L,), -1, jnp.int32)

    def init_step(t, carry):
        winner[pl.ds(t * L, L)] = minus1
        return carry

    lax.fori_loop(0, 2 * NV, init_step, 0)

    def build_step(k, carry):
        ids = idx_all[pl.ds(k * L, L)]
        p, half = _wrow(ids)
        m = (p >= lo) & (p < lo + WRNG)

        @pl.when(jnp.any(m))
        def _():
            pos = iota + k * L
            rel = jnp.where(m, (p - lo) + half * WRNG, 0)

            def body(keep_going):
                cur = plsc.load_gather(winner, [rel], mask=m)
                plsc.store_scatter(winner, [rel], pos, mask=m & (cur < pos))
                chk = plsc.load_gather(winner, [rel], mask=m)
                return jnp.any(m & (chk < pos))

            lax.while_loop(lambda kg: kg, body, True)

        return carry

    lax.fori_loop(0, B // L, build_step, 0)

    def make_compact(pos_buf, row_buf, base_t):
        def compact_step(t, cnt):
            w = winner[pl.ds((base_t + t) * L, L)]
            m = w >= 0
            c = jnp.sum(m.astype(jnp.int32))

            @pl.when((c > 0) & (cnt <= SEL - L))
            def _():
                plsc.store_compressed(pos_buf.at[pl.ds(cnt, L)], w, mask=m)
                rows = iota + (lo + t * L)
                plsc.store_compressed(row_buf.at[pl.ds(cnt, L)], rows,
                                      mask=m)

            return jnp.minimum(cnt + c, SEL - L)

        return compact_step

    cntA = lax.fori_loop(0, NV, make_compact(posA, rowA, 0), 0)
    cntB = lax.fori_loop(0, NV, make_compact(posB, rowB, NV), 0)

    def pad_tail(pos_buf, row_buf, cnt):
        @pl.when(cnt > 0)
        def _():
            zero16 = jnp.zeros((L,), jnp.int32)
            e_r = row_buf[pl.ds(0, L)].at[zero16].get(
                mode="promise_in_bounds")
            e_p = pos_buf[pl.ds(0, L)].at[zero16].get(
                mode="promise_in_bounds")
            row_buf[pl.ds(cnt, L)] = e_r
            pos_buf[pl.ds(cnt, L)] = e_p

    pad_tail(posA, rowA, cntA)
    pad_tail(posB, rowB, cntB)

    def make_phase(pos_buf, row_buf, lane0):
        def step(c2, carry):
            pos_v = pos_buf[pl.ds(c2 * SCH, SCH)]
            row_v = row_buf[pl.ds(c2 * SCH, SCH)]
            cp_n = pltpu.async_copy(newh_hbm.at[pos_v], nbuf, gsem)
            cp_w = pltpu.async_copy(w_ref.at[row_v], wbuf, gsem)
            cp_n.wait()
            cp_w.wait()
            for r in range(SCH):
                for g in range(H // L):
                    wbuf[r, pl.ds(lane0 + g * L, L)] = (
                        nbuf[r, pl.ds(g * L, L)])
            pltpu.async_copy(wbuf, w_ref.at[row_v], ssem).wait()
            return carry

        return step

    nchA = (cntA + SCH - 1) // SCH
    lax.fori_loop(0, nchA, make_phase(posA, rowA, 0), 0)
    nchB = (cntB + SCH - 1) // SCH
    lax.fori_loop(0, nchB, make_phase(posB, rowB, H), 0)


@functools.cache
def _sc_scatter_kernel():
    return pl.kernel(
        _sc_scatter_body,
        mesh=_mesh(),
        out_type=(),
        scratch_types=[
            pltpu.VMEM((B,), jnp.int32),
            pltpu.VMEM((2 * WRNG,), jnp.int32),
            pltpu.VMEM((SEL,), jnp.int32),
            pltpu.VMEM((SEL,), jnp.int32),
            pltpu.VMEM((SEL,), jnp.int32),
            pltpu.VMEM((SEL,), jnp.int32),
            pltpu.VMEM((SCH, WD), jnp.float32),
            pltpu.VMEM((SCH, WD), jnp.float32),
            pltpu.SemaphoreType.DMA,
            pltpu.SemaphoreType.DMA,
        ],
        compiler_params=pltpu.CompilerParams(needs_layout_passes=False),
    )


def kernel(features, node_ids, hidden_state, W_ih, W_hh, b_ih, b_hh, W_out,
           b_out):
    ids = node_ids.astype(jnp.int32)

    w_table = _tc_pack(hidden_state.T)
    prev_pair = _sc_gather_kernel()(w_table, ids)

    wir, wiz, win = (W_ih[0:H].T, W_ih[H:2 * H].T, W_ih[2 * H:].T)
    whr, whz, whn = (W_hh[0:H].T, W_hh[H:2 * H].T, W_hh[2 * H:].T)
    br = (b_ih[0:H] + b_hh[0:H]).reshape(1, H)
    bz = (b_ih[H:2 * H] + b_hh[H:2 * H]).reshape(1, H)
    bin_ = b_ih[2 * H:].reshape(1, H)
    bhn = b_hh[2 * H:].reshape(1, H)

    new_h_pad, logits = _tc_gru(features, prev_pair, ids.reshape(B, 1), wir,
                                wiz, win, whr, whz, whn, br, bz, bin_, bhn,
                                W_out.T, b_out.reshape(1, C))

    w_ref = jax.new_ref(w_table)
    _sc_scatter_kernel()(ids, new_h_pad, w_ref)
    return logits, _tc_unpack(w_ref[...]).T
